# Initial kernel scaffold; baseline (speedup 1.0000x reference)
import jax
import jax.numpy as jnp
from jax.experimental import pallas as pl


def kernel(xyz, points, W0, b0, g0, be0, W1, b1, g1, be1, W2, b2, g2, be2):
    raise NotImplementedError("write your pallas kernel here")


# speedup vs baseline: 9.3932x; 9.3932x over previous
import functools

import jax
import jax.numpy as jnp
from jax import lax
from jax.experimental import pallas as pl
from jax.experimental.pallas import tpu as pltpu
from jax.experimental.pallas import tpu_sc as plsc

B, N, D = 4, 4096, 16
NPOINT, NSAMPLE = 1024, 32
GROUPS = 8
EPS = 1e-5
CIN = 128
CHUNK_C = 128
ROWS = B * NPOINT * NSAMPLE
RCHUNK = NPOINT * NSAMPLE // 8
NCHUNK = ROWS // RCHUNK


def _fps_body(xyzt_ref, f0_ref, nxyz_ref):
    X = xyzt_ref[:, 0, :].reshape(B, 32, 128)
    Y = xyzt_ref[:, 1, :].reshape(B, 32, 128)
    Z = xyzt_ref[:, 2, :].reshape(B, 32, 128)
    s32 = lax.broadcasted_iota(jnp.int32, (B, 32, 128), 1)
    lane = lax.broadcasted_iota(jnp.int32, (B, 32, 128), 2)
    flat = s32 * 128 + lane
    tkey = (((s32 % 8) * 128 + lane) * 4 + s32 // 8) * 4096 + flat

    def step(i, carry):
        dist, fi = carry
        eq = flat == fi
        cx = jnp.sum(jnp.where(eq, X, 0.0), axis=(1, 2), keepdims=True)
        cy = jnp.sum(jnp.where(eq, Y, 0.0), axis=(1, 2), keepdims=True)
        cz = jnp.sum(jnp.where(eq, Z, 0.0), axis=(1, 2), keepdims=True)
        nxyz_ref[:, pl.ds(i, 1), :] = jnp.concatenate(
            [cx[:, :, 0], cy[:, :, 0], cz[:, :, 0]], axis=1)[:, None, :]
        dx = X - cx
        dy = Y - cy
        dz = Z - cz
        d = dx * dx + dy * dy + dz * dz
        dist = jnp.minimum(dist, d)
        m = jnp.max(dist, axis=(1, 2), keepdims=True)
        k = jnp.min(jnp.where(dist == m, tkey, jnp.int32(2 ** 30)),
                    axis=(1, 2), keepdims=True)
        return dist, lax.bitwise_and(k, jnp.int32(4095))

    dist0 = jnp.full((B, 32, 128), 1e10, jnp.float32)
    fi0 = f0_ref[:, 0:1].reshape(B, 1, 1)
    lax.fori_loop(0, NPOINT, step, (dist0, fi0))


def _fps(xyzt, f0):
    return pl.pallas_call(
        _fps_body,
        out_shape=jax.ShapeDtypeStruct((B, NPOINT, 3), jnp.float32),
        in_specs=[
            pl.BlockSpec((B, 3, N), lambda: (0, 0, 0)),
            pl.BlockSpec((B, 128), lambda: (0, 0)),
        ],
        out_specs=pl.BlockSpec((B, NPOINT, 3), lambda: (0, 0, 0)),
    )(xyzt, f0)


def _knn_body(nxyz_ref, xyzt_ref, idx_ref):
    b = pl.program_id(0)
    nx = nxyz_ref[0]
    xt = xyzt_ref[0]
    s = jnp.dot(nx.astype(jnp.bfloat16), xt.astype(jnp.bfloat16),
                preferred_element_type=jnp.float32)
    a2 = (nx[:, 0:1] * nx[:, 0:1] + nx[:, 1:2] * nx[:, 1:2]) + nx[:, 2:3] * nx[:, 2:3]
    b2 = (xt[0:1] * xt[0:1] + xt[1:2] * xt[1:2]) + xt[2:3] * xt[2:3]
    d = (-2.0 * s + a2) + b2
    col = lax.broadcasted_iota(jnp.int32, (CHUNK_C, N), 1)
    base = b * N
    inf = jnp.float32(jnp.inf)
    for s in range(NSAMPLE):
        m = jnp.min(d, axis=1, keepdims=True)
        ci = jnp.min(jnp.where(d == m, col, jnp.int32(2 ** 30)),
                     axis=1, keepdims=True)
        idx_ref[0, :, pl.ds(s, 1)] = ci + base
        d = jnp.where(col == ci, inf, d)


def _knn(nxyz, xyzt):
    return pl.pallas_call(
        _knn_body,
        grid=(B, NPOINT // CHUNK_C),
        out_shape=jax.ShapeDtypeStruct((B, NPOINT, NSAMPLE), jnp.int32),
        in_specs=[
            pl.BlockSpec((1, CHUNK_C, 3), lambda b, c: (b, c, 0)),
            pl.BlockSpec((1, 3, N), lambda b, c: (b, 0, 0)),
        ],
        out_specs=pl.BlockSpec((1, CHUNK_C, NSAMPLE), lambda b, c: (b, c, 0)),
    )(nxyz, xyzt)


@functools.lru_cache(maxsize=1)
def _make_gather():
    info = plsc.get_sparse_core_info()
    nc, ns = info.num_cores, info.num_subcores
    nw = nc * ns
    rpw = ROWS // nw
    gchunk = 128
    niter = rpw // gchunk
    mesh = plsc.VectorSubcoreMesh(core_axis_name="c", subcore_axis_name="s")

    @functools.partial(
        pl.kernel, mesh=mesh,
        out_type=jax.ShapeDtypeStruct((ROWS, CIN), jnp.float32),
        scratch_types=[
            pltpu.VMEM((gchunk,), jnp.int32),
            pltpu.VMEM((gchunk, CIN), jnp.float32),
            pltpu.SemaphoreType.DMA,
        ],
    )
    def gk(table_hbm, idx_hbm, out_hbm, idx_v, rows_v, sem):
        wid = lax.axis_index("s") * nc + lax.axis_index("c")
        base = wid * rpw

        def body(c, _):
            off = base + c * gchunk
            pltpu.sync_copy(idx_hbm.at[pl.ds(off, gchunk)], idx_v)
            pltpu.async_copy(table_hbm.at[idx_v], rows_v, sem).wait()
            pltpu.sync_copy(rows_v, out_hbm.at[pl.ds(off, gchunk)])
            return 0

        lax.fori_loop(0, niter, body, 0)

    return gk


def _gather_rows(table, idx_flat):
    return _make_gather()(table, idx_flat)


def _gnorm_coef(m_ref, gam_ref, bet_ref, cout, bidx):
    per_g = cout // GROUPS
    cnt = jnp.float32(per_g * NPOINT * NSAMPLE)
    sums = jnp.sum(m_ref[pl.ds(bidx * (NCHUNK // B), NCHUNK // B)], axis=0)
    s = sums[0:1, :]
    q = sums[1:2, :]
    ci = lax.broadcasted_iota(jnp.int32, (cout, cout), 0) // per_g
    cj = lax.broadcasted_iota(jnp.int32, (cout, cout), 1) // per_g
    A = (ci == cj).astype(jnp.float32)
    gs = jnp.dot(s, A, preferred_element_type=jnp.float32)
    gq = jnp.dot(q, A, preferred_element_type=jnp.float32)
    mean = gs / cnt
    var = gq / cnt - mean * mean
    inv = lax.rsqrt(var + EPS)
    scale = inv * gam_ref[...]
    shift = bet_ref[...] - mean * scale
    return scale, shift


def _p1_body(g_ref, nx_ref, w_ref, b_ref, z_ref, m_ref):
    g = g_ref[...].reshape(CHUNK_C, NSAMPLE, CIN)
    gn = (g - nx_ref[...][:, None, :]).reshape(RCHUNK, CIN)
    z = jnp.dot(gn, w_ref[...], preferred_element_type=jnp.float32) + b_ref[...]
    z_ref[...] = z
    m_ref[0, 0:1, :] = jnp.sum(z, axis=0, keepdims=True)
    m_ref[0, 1:2, :] = jnp.sum(z * z, axis=0, keepdims=True)


def _p23_body(z_ref, m_ref, gam_ref, bet_ref, w_ref, b_ref, zo_ref, mo_ref,
              *, cin, cout):
    bidx = pl.program_id(0) // (NCHUNK // B)
    scale, shift = _gnorm_coef(m_ref, gam_ref, bet_ref, cin, bidx)
    a = jnp.maximum(z_ref[...] * scale + shift, 0.0)
    z = jnp.dot(a, w_ref[...], preferred_element_type=jnp.float32) + b_ref[...]
    zo_ref[...] = z
    mo_ref[0, 0:1, :] = jnp.sum(z, axis=0, keepdims=True)
    mo_ref[0, 1:2, :] = jnp.sum(z * z, axis=0, keepdims=True)


def _p4_body(z_ref, m_ref, gam_ref, bet_ref, out_ref, *, cout):
    bidx = pl.program_id(0) // (NCHUNK // B)
    scale, shift = _gnorm_coef(m_ref, gam_ref, bet_ref, cout, bidx)
    a = jnp.maximum(z_ref[...] * scale + shift, 0.0)
    out_ref[0] = jnp.max(a.reshape(CHUNK_C, NSAMPLE, cout), axis=1)


def _mlp(gath, nxp, W0p, b0r, g0r, be0r, W1t, b1r, g1r, be1r,
         W2t, b2r, g2r, be2r):
    f32 = jnp.float32
    zspec = lambda c: pl.BlockSpec((RCHUNK, c), lambda i: (i, 0))
    mspec_o = lambda c: pl.BlockSpec((1, 2, c), lambda i: (i, 0, 0))
    mspec_i = lambda c: pl.BlockSpec((NCHUNK, 2, c), lambda i: (0, 0, 0))
    wspec = lambda r, c: pl.BlockSpec((r, c), lambda i: (0, 0))

    z1, m1 = pl.pallas_call(
        _p1_body,
        grid=(NCHUNK,),
        out_shape=(jax.ShapeDtypeStruct((ROWS, 64), f32),
                   jax.ShapeDtypeStruct((NCHUNK, 2, 64), f32)),
        in_specs=[zspec(CIN),
                  pl.BlockSpec((CHUNK_C, CIN), lambda i: (i, 0)),
                  wspec(CIN, 64), wspec(1, 64)],
        out_specs=(zspec(64), mspec_o(64)),
    )(gath, nxp, W0p, b0r)

    z2, m2 = pl.pallas_call(
        functools.partial(_p23_body, cin=64, cout=64),
        grid=(NCHUNK,),
        out_shape=(jax.ShapeDtypeStruct((ROWS, 64), f32),
                   jax.ShapeDtypeStruct((NCHUNK, 2, 64), f32)),
        in_specs=[zspec(64), mspec_i(64), wspec(1, 64), wspec(1, 64),
                  wspec(64, 64), wspec(1, 64)],
        out_specs=(zspec(64), mspec_o(64)),
    )(z1, m1, g0r, be0r, W1t, b1r)

    z3, m3 = pl.pallas_call(
        functools.partial(_p23_body, cin=64, cout=128),
        grid=(NCHUNK,),
        out_shape=(jax.ShapeDtypeStruct((ROWS, 128), f32),
                   jax.ShapeDtypeStruct((NCHUNK, 2, 128), f32)),
        in_specs=[zspec(64), mspec_i(64), wspec(1, 64), wspec(1, 64),
                  wspec(64, 128), wspec(1, 128)],
        out_specs=(zspec(128), mspec_o(128)),
    )(z2, m2, g1r, be1r, W2t, b2r)

    out = pl.pallas_call(
        functools.partial(_p4_body, cout=128),
        grid=(NCHUNK,),
        out_shape=jax.ShapeDtypeStruct((B, NPOINT, 128), f32),
        in_specs=[zspec(128), mspec_i(128), wspec(1, 128), wspec(1, 128)],
        out_specs=pl.BlockSpec((1, CHUNK_C, 128),
                               lambda i: (i // (NCHUNK // B), i % (NCHUNK // B), 0)),
    )(z3, m3, g2r, be2r)
    return out


def kernel(xyz, points, W0, b0, g0, be0, W1, b1, g1, be1, W2, b2, g2, be2):
    f32 = jnp.float32
    xyzt = xyz.transpose(0, 2, 1)
    f0 = jax.random.randint(jax.random.key(42), (B,), 0, N)
    f0 = jnp.broadcast_to(f0.astype(jnp.int32)[:, None], (B, 128))

    new_xyz = _fps(xyzt, f0)
    idx = _knn(new_xyz, xyzt)

    table = jnp.concatenate([xyz, points], axis=-1).reshape(B * N, 3 + D)
    table = jnp.pad(table, ((0, 0), (0, CIN - (3 + D))))
    gath = _gather_rows(table, idx.reshape(-1))

    nxp = jnp.pad(new_xyz.reshape(B * NPOINT, 3), ((0, 0), (0, CIN - 3)))
    W0p = jnp.pad(W0.T, ((0, CIN - (3 + D)), (0, 0)))
    out = _mlp(gath, nxp, W0p, b0[None, :], g0[None, :], be0[None, :],
               W1.T, b1[None, :], g1[None, :], be1[None, :],
               W2.T, b2[None, :], g2[None, :], be2[None, :])
    return (new_xyz, out)


# speedup vs baseline: 11.4519x; 1.2192x over previous
import functools

import jax
import jax.numpy as jnp
from jax import lax
from jax.experimental import pallas as pl
from jax.experimental.pallas import tpu as pltpu
from jax.experimental.pallas import tpu_sc as plsc

B, N, D = 4, 4096, 16
NPOINT, NSAMPLE = 1024, 32
GROUPS = 8
EPS = 1e-5
CIN = 128
CHUNK_C = 128
KNN_C = 512
ROWS = B * NPOINT * NSAMPLE
RCHUNK = NPOINT * NSAMPLE // 8
NCHUNK = ROWS // RCHUNK


def _fps_body(xyzt_ref, f0_ref, nxyz_ref):
    XYZ = xyzt_ref[...].reshape(B, 3, 32, 128)
    X = XYZ[:, 0]
    Y = XYZ[:, 1]
    Z = XYZ[:, 2]
    s32 = lax.broadcasted_iota(jnp.int32, (B, 32, 128), 1)
    lane = lax.broadcasted_iota(jnp.int32, (B, 32, 128), 2)
    flat = s32 * 128 + lane
    tkey = (((s32 % 8) * 128 + lane) * 4 + s32 // 8) * 4096 + flat

    def step(i, carry):
        dist, fi = carry
        eq = flat == fi
        c3 = jnp.sum(jnp.where(eq[:, None], XYZ, 0.0), axis=(2, 3))
        nxyz_ref[:, pl.ds(i, 1), :] = c3[:, None, :]
        dx = X - c3[:, 0:1, None]
        dy = Y - c3[:, 1:2, None]
        dz = Z - c3[:, 2:3, None]
        d = dx * dx + dy * dy + dz * dz
        dist = jnp.minimum(dist, d)
        m = jnp.max(dist, axis=(1, 2), keepdims=True)
        k = jnp.min(jnp.where(dist == m, tkey, jnp.int32(2 ** 30)),
                    axis=(1, 2), keepdims=True)
        return dist, lax.bitwise_and(k, jnp.int32(4095))

    dist0 = jnp.full((B, 32, 128), 1e10, jnp.float32)
    fi0 = f0_ref[:, 0:1].reshape(B, 1, 1)
    lax.fori_loop(0, NPOINT, step, (dist0, fi0))


def _fps(xyzt, f0):
    return pl.pallas_call(
        _fps_body,
        out_shape=jax.ShapeDtypeStruct((B, NPOINT, 3), jnp.float32),
        in_specs=[
            pl.BlockSpec((B, 3, N), lambda: (0, 0, 0)),
            pl.BlockSpec((B, 128), lambda: (0, 0)),
        ],
        out_specs=pl.BlockSpec((B, NPOINT, 3), lambda: (0, 0, 0)),
    )(xyzt, f0)


def _knn_body(nxyz_ref, xyzt_ref, idx_ref):
    b = pl.program_id(0)
    nx = nxyz_ref[0]
    xt = xyzt_ref[0]
    s = jnp.dot(nx.astype(jnp.bfloat16), xt.astype(jnp.bfloat16),
                preferred_element_type=jnp.float32)
    a2 = (nx[:, 0:1] * nx[:, 0:1] + nx[:, 1:2] * nx[:, 1:2]) + nx[:, 2:3] * nx[:, 2:3]
    b2 = (xt[0:1] * xt[0:1] + xt[1:2] * xt[1:2]) + xt[2:3] * xt[2:3]
    d = (-2.0 * s + a2) + b2
    col = lax.broadcasted_iota(jnp.int32, (KNN_C, N), 1)
    base = b * N
    inf = jnp.float32(jnp.inf)
    for s in range(NSAMPLE):
        m = jnp.min(d, axis=1, keepdims=True)
        ci = jnp.min(jnp.where(d == m, col, jnp.int32(2 ** 30)),
                     axis=1, keepdims=True)
        idx_ref[0, :, pl.ds(s, 1)] = ci + base
        d = jnp.where(col == ci, inf, d)


def _knn(nxyz, xyzt):
    return pl.pallas_call(
        _knn_body,
        grid=(B, NPOINT // KNN_C),
        out_shape=jax.ShapeDtypeStruct((B, NPOINT, NSAMPLE), jnp.int32),
        in_specs=[
            pl.BlockSpec((1, KNN_C, 3), lambda b, c: (b, c, 0)),
            pl.BlockSpec((1, 3, N), lambda b, c: (b, 0, 0)),
        ],
        out_specs=pl.BlockSpec((1, KNN_C, NSAMPLE), lambda b, c: (b, c, 0)),
    )(nxyz, xyzt)


@functools.lru_cache(maxsize=1)
def _make_gather():
    info = plsc.get_sparse_core_info()
    nc, ns = info.num_cores, info.num_subcores
    nw = nc * ns
    rpw = ROWS // nw
    gchunk = 128
    niter = rpw // gchunk
    mesh = plsc.VectorSubcoreMesh(core_axis_name="c", subcore_axis_name="s")

    @functools.partial(
        pl.kernel, mesh=mesh,
        out_type=jax.ShapeDtypeStruct((ROWS, CIN), jnp.float32),
        scratch_types=[
            pltpu.VMEM((gchunk,), jnp.int32),
            pltpu.VMEM((gchunk, CIN), jnp.float32),
            pltpu.SemaphoreType.DMA,
        ],
    )
    def gk(table_hbm, idx_hbm, out_hbm, idx_v, rows_v, sem):
        wid = lax.axis_index("s") * nc + lax.axis_index("c")
        base = wid * rpw

        def body(c, _):
            off = base + c * gchunk
            pltpu.sync_copy(idx_hbm.at[pl.ds(off, gchunk)], idx_v)
            pltpu.async_copy(table_hbm.at[idx_v], rows_v, sem).wait()
            pltpu.sync_copy(rows_v, out_hbm.at[pl.ds(off, gchunk)])
            return 0

        lax.fori_loop(0, niter, body, 0)

    return gk


def _gather_rows(table, idx_flat):
    return _make_gather()(table, idx_flat)


def _gnorm_coef(m_ref, gam_ref, bet_ref, cout, bidx):
    per_g = cout // GROUPS
    cnt = jnp.float32(per_g * NPOINT * NSAMPLE)
    sums = jnp.sum(m_ref[pl.ds(bidx * (NCHUNK // B), NCHUNK // B)], axis=0)
    s = sums[0:1, :]
    q = sums[1:2, :]
    ci = lax.broadcasted_iota(jnp.int32, (cout, cout), 0) // per_g
    cj = lax.broadcasted_iota(jnp.int32, (cout, cout), 1) // per_g
    A = (ci == cj).astype(jnp.float32)
    gs = jnp.dot(s, A, preferred_element_type=jnp.float32)
    gq = jnp.dot(q, A, preferred_element_type=jnp.float32)
    mean = gs / cnt
    var = gq / cnt - mean * mean
    inv = lax.rsqrt(var + EPS)
    scale = inv * gam_ref[...]
    shift = bet_ref[...] - mean * scale
    return scale, shift


def _p1_body(g_ref, nx_ref, w_ref, b_ref, z_ref, m_ref):
    g = g_ref[...].reshape(CHUNK_C, NSAMPLE, CIN)
    gn = (g - nx_ref[...][:, None, :]).reshape(RCHUNK, CIN)
    z = jnp.dot(gn, w_ref[...], preferred_element_type=jnp.float32) + b_ref[...]
    z_ref[...] = z
    m_ref[0, 0:1, :] = jnp.sum(z, axis=0, keepdims=True)
    m_ref[0, 1:2, :] = jnp.sum(z * z, axis=0, keepdims=True)


def _p23_body(z_ref, m_ref, gam_ref, bet_ref, w_ref, b_ref, zo_ref, mo_ref,
              *, cin, cout):
    bidx = pl.program_id(0) // (NCHUNK // B)
    scale, shift = _gnorm_coef(m_ref, gam_ref, bet_ref, cin, bidx)
    a = jnp.maximum(z_ref[...] * scale + shift, 0.0)
    z = jnp.dot(a, w_ref[...], preferred_element_type=jnp.float32) + b_ref[...]
    zo_ref[...] = z
    mo_ref[0, 0:1, :] = jnp.sum(z, axis=0, keepdims=True)
    mo_ref[0, 1:2, :] = jnp.sum(z * z, axis=0, keepdims=True)


def _p4_body(z_ref, m_ref, gam_ref, bet_ref, out_ref, *, cout):
    bidx = pl.program_id(0) // (NCHUNK // B)
    scale, shift = _gnorm_coef(m_ref, gam_ref, bet_ref, cout, bidx)
    a = jnp.maximum(z_ref[...] * scale + shift, 0.0)
    out_ref[0] = jnp.max(a.reshape(CHUNK_C, NSAMPLE, cout), axis=1)


def _mlp(gath, nxp, W0p, b0r, g0r, be0r, W1t, b1r, g1r, be1r,
         W2t, b2r, g2r, be2r):
    f32 = jnp.float32
    zspec = lambda c: pl.BlockSpec((RCHUNK, c), lambda i: (i, 0))
    mspec_o = lambda c: pl.BlockSpec((1, 2, c), lambda i: (i, 0, 0))
    mspec_i = lambda c: pl.BlockSpec((NCHUNK, 2, c), lambda i: (0, 0, 0))
    wspec = lambda r, c: pl.BlockSpec((r, c), lambda i: (0, 0))

    z1, m1 = pl.pallas_call(
        _p1_body,
        grid=(NCHUNK,),
        out_shape=(jax.ShapeDtypeStruct((ROWS, 64), f32),
                   jax.ShapeDtypeStruct((NCHUNK, 2, 64), f32)),
        in_specs=[zspec(CIN),
                  pl.BlockSpec((CHUNK_C, CIN), lambda i: (i, 0)),
                  wspec(CIN, 64), wspec(1, 64)],
        out_specs=(zspec(64), mspec_o(64)),
    )(gath, nxp, W0p, b0r)

    z2, m2 = pl.pallas_call(
        functools.partial(_p23_body, cin=64, cout=64),
        grid=(NCHUNK,),
        out_shape=(jax.ShapeDtypeStruct((ROWS, 64), f32),
                   jax.ShapeDtypeStruct((NCHUNK, 2, 64), f32)),
        in_specs=[zspec(64), mspec_i(64), wspec(1, 64), wspec(1, 64),
                  wspec(64, 64), wspec(1, 64)],
        out_specs=(zspec(64), mspec_o(64)),
    )(z1, m1, g0r, be0r, W1t, b1r)

    z3, m3 = pl.pallas_call(
        functools.partial(_p23_body, cin=64, cout=128),
        grid=(NCHUNK,),
        out_shape=(jax.ShapeDtypeStruct((ROWS, 128), f32),
                   jax.ShapeDtypeStruct((NCHUNK, 2, 128), f32)),
        in_specs=[zspec(64), mspec_i(64), wspec(1, 64), wspec(1, 64),
                  wspec(64, 128), wspec(1, 128)],
        out_specs=(zspec(128), mspec_o(128)),
    )(z2, m2, g1r, be1r, W2t, b2r)

    out = pl.pallas_call(
        functools.partial(_p4_body, cout=128),
        grid=(NCHUNK,),
        out_shape=jax.ShapeDtypeStruct((B, NPOINT, 128), f32),
        in_specs=[zspec(128), mspec_i(128), wspec(1, 128), wspec(1, 128)],
        out_specs=pl.BlockSpec((1, CHUNK_C, 128),
                               lambda i: (i // (NCHUNK // B), i % (NCHUNK // B), 0)),
    )(z3, m3, g2r, be2r)
    return out


def kernel(xyz, points, W0, b0, g0, be0, W1, b1, g1, be1, W2, b2, g2, be2):
    f32 = jnp.float32
    xyzt = xyz.transpose(0, 2, 1)
    f0 = jax.random.randint(jax.random.key(42), (B,), 0, N)
    f0 = jnp.broadcast_to(f0.astype(jnp.int32)[:, None], (B, 128))

    new_xyz = _fps(xyzt, f0)
    idx = _knn(new_xyz, xyzt)

    table = jnp.concatenate([xyz, points], axis=-1).reshape(B * N, 3 + D)
    table = jnp.pad(table, ((0, 0), (0, CIN - (3 + D))))
    gath = _gather_rows(table, idx.reshape(-1))

    nxp = jnp.pad(new_xyz.reshape(B * NPOINT, 3), ((0, 0), (0, CIN - 3)))
    W0p = jnp.pad(W0.T, ((0, CIN - (3 + D)), (0, 0)))
    out = _mlp(gath, nxp, W0p, b0[None, :], g0[None, :], be0[None, :],
               W1.T, b1[None, :], g1[None, :], be1[None, :],
               W2.T, b2[None, :], g2[None, :], be2[None, :])
    return (new_xyz, out)


# speedup vs baseline: 11.4807x; 1.0025x over previous
import functools

import jax
import jax.numpy as jnp
from jax import lax
from jax.experimental import pallas as pl
from jax.experimental.pallas import tpu as pltpu
from jax.experimental.pallas import tpu_sc as plsc

B, N, D = 4, 4096, 16
NPOINT, NSAMPLE = 1024, 32
GROUPS = 8
EPS = 1e-5
CIN = 128
CHUNK_C = 128
KNN_C = 512
ROWS = B * NPOINT * NSAMPLE
RCHUNK = NPOINT * NSAMPLE // 8
NCHUNK = ROWS // RCHUNK


def _fps_body(xyzt_ref, f0_ref, nxyz_ref):
    XYZ = xyzt_ref[...].reshape(B, 3, 32, 128)
    X = XYZ[:, 0]
    Y = XYZ[:, 1]
    Z = XYZ[:, 2]
    s32 = lax.broadcasted_iota(jnp.int32, (B, 32, 128), 1)
    lane = lax.broadcasted_iota(jnp.int32, (B, 32, 128), 2)
    flat = s32 * 128 + lane
    tkey = (((s32 % 8) * 128 + lane) * 4 + s32 // 8) * 4096 + flat

    def step(i, carry):
        dist, fi = carry
        eq = flat == fi
        c3 = jnp.sum(jnp.where(eq[:, None], XYZ, 0.0), axis=(2, 3))
        nxyz_ref[:, pl.ds(i, 1), :] = c3[:, None, :]
        dx = X - c3[:, 0:1, None]
        dy = Y - c3[:, 1:2, None]
        dz = Z - c3[:, 2:3, None]
        d = dx * dx + dy * dy + dz * dz
        dist = jnp.minimum(dist, d)
        m = jnp.max(dist, axis=(1, 2), keepdims=True)
        k = jnp.min(jnp.where(dist == m, tkey, jnp.int32(2 ** 30)),
                    axis=(1, 2), keepdims=True)
        return dist, lax.bitwise_and(k, jnp.int32(4095))

    dist0 = jnp.full((B, 32, 128), 1e10, jnp.float32)
    fi0 = f0_ref[:, 0:1].reshape(B, 1, 1)
    lax.fori_loop(0, NPOINT, step, (dist0, fi0))


def _fps(xyzt, f0):
    return pl.pallas_call(
        _fps_body,
        out_shape=jax.ShapeDtypeStruct((B, NPOINT, 3), jnp.float32),
        in_specs=[
            pl.BlockSpec((B, 3, N), lambda: (0, 0, 0)),
            pl.BlockSpec((B, 128), lambda: (0, 0)),
        ],
        out_specs=pl.BlockSpec((B, NPOINT, 3), lambda: (0, 0, 0)),
    )(xyzt, f0)


def _knn_body(nxyz_ref, xyzt_ref, idx_ref):
    b = pl.program_id(0)
    nx = nxyz_ref[0]
    xt = xyzt_ref[0]
    s = jnp.dot(nx.astype(jnp.bfloat16), xt.astype(jnp.bfloat16),
                preferred_element_type=jnp.float32)
    a2 = (nx[:, 0:1] * nx[:, 0:1] + nx[:, 1:2] * nx[:, 1:2]) + nx[:, 2:3] * nx[:, 2:3]
    b2 = (xt[0:1] * xt[0:1] + xt[1:2] * xt[1:2]) + xt[2:3] * xt[2:3]
    d = (-2.0 * s + a2) + b2
    col = lax.broadcasted_iota(jnp.int32, (KNN_C, N), 1)
    base = b * N
    inf = jnp.float32(jnp.inf)
    for s in range(NSAMPLE):
        m = jnp.min(d, axis=1, keepdims=True)
        ci = jnp.min(jnp.where(d == m, col, jnp.int32(2 ** 30)),
                     axis=1, keepdims=True)
        idx_ref[0, :, pl.ds(s, 1)] = ci + base
        d = jnp.where(col == ci, inf, d)


def _knn(nxyz, xyzt):
    return pl.pallas_call(
        _knn_body,
        grid=(B, NPOINT // KNN_C),
        out_shape=jax.ShapeDtypeStruct((B, NPOINT, NSAMPLE), jnp.int32),
        in_specs=[
            pl.BlockSpec((1, KNN_C, 3), lambda b, c: (b, c, 0)),
            pl.BlockSpec((1, 3, N), lambda b, c: (b, 0, 0)),
        ],
        out_specs=pl.BlockSpec((1, KNN_C, NSAMPLE), lambda b, c: (b, c, 0)),
    )(nxyz, xyzt)


@functools.lru_cache(maxsize=1)
def _make_gather():
    info = plsc.get_sparse_core_info()
    nc, ns = info.num_cores, info.num_subcores
    nw = nc * ns
    rpw = ROWS // nw
    gchunk = 128
    niter = rpw // gchunk
    mesh = plsc.VectorSubcoreMesh(core_axis_name="c", subcore_axis_name="s")

    @functools.partial(
        pl.kernel, mesh=mesh,
        out_type=jax.ShapeDtypeStruct((ROWS, CIN), jnp.float32),
        scratch_types=[
            pltpu.VMEM((gchunk,), jnp.int32),
            pltpu.VMEM((gchunk, CIN), jnp.float32),
            pltpu.SemaphoreType.DMA,
        ],
    )
    def gk(table_hbm, idx_hbm, out_hbm, idx_v, rows_v, sem):
        wid = lax.axis_index("s") * nc + lax.axis_index("c")
        base = wid * rpw

        def body(c, _):
            off = base + c * gchunk
            pltpu.sync_copy(idx_hbm.at[pl.ds(off, gchunk)], idx_v)
            pltpu.async_copy(table_hbm.at[idx_v], rows_v, sem).wait()
            pltpu.sync_copy(rows_v, out_hbm.at[pl.ds(off, gchunk)])
            return 0

        lax.fori_loop(0, niter, body, 0)

    return gk


def _gather_rows(table, idx_flat):
    return _make_gather()(table, idx_flat)


def _gnorm_coef(m_ref, gam_ref, bet_ref, cout, bidx):
    per_g = cout // GROUPS
    cnt = jnp.float32(per_g * NPOINT * NSAMPLE)
    sums = jnp.sum(m_ref[pl.ds(bidx * (NCHUNK // B), NCHUNK // B)], axis=0)
    s = sums[0:1, :]
    q = sums[1:2, :]
    ci = lax.broadcasted_iota(jnp.int32, (cout, cout), 0) // per_g
    cj = lax.broadcasted_iota(jnp.int32, (cout, cout), 1) // per_g
    A = (ci == cj).astype(jnp.float32)
    gs = jnp.dot(s, A, preferred_element_type=jnp.float32)
    gq = jnp.dot(q, A, preferred_element_type=jnp.float32)
    mean = gs / cnt
    var = gq / cnt - mean * mean
    inv = lax.rsqrt(var + EPS)
    scale = inv * gam_ref[...]
    shift = bet_ref[...] - mean * scale
    return scale, shift


def _p1_body(g_ref, nx_ref, w_ref, b_ref, z_ref, m_ref):
    g = g_ref[...].reshape(CHUNK_C, NSAMPLE, CIN)
    gn = (g - nx_ref[...][:, None, :]).reshape(RCHUNK, CIN)
    z = jnp.dot(gn, w_ref[...], preferred_element_type=jnp.float32) + b_ref[...]
    z_ref[...] = z
    m_ref[0, 0:1, :] = jnp.sum(z, axis=0, keepdims=True)
    m_ref[0, 1:2, :] = jnp.sum(z * z, axis=0, keepdims=True)


def _p23_body(z_ref, m_ref, gam_ref, bet_ref, w_ref, b_ref, zo_ref, mo_ref,
              *, cin, cout):
    bidx = pl.program_id(0) // (NCHUNK // B)
    scale, shift = _gnorm_coef(m_ref, gam_ref, bet_ref, cin, bidx)
    a = jnp.maximum(z_ref[...] * scale + shift, 0.0)
    z = jnp.dot(a, w_ref[...], preferred_element_type=jnp.float32) + b_ref[...]
    zo_ref[...] = z
    mo_ref[0, 0:1, :] = jnp.sum(z, axis=0, keepdims=True)
    mo_ref[0, 1:2, :] = jnp.sum(z * z, axis=0, keepdims=True)


def _p3_body(z_ref, m_ref, gam_ref, bet_ref, w_ref, b_ref, mo_ref, *, cin):
    bidx = pl.program_id(0) // (NCHUNK // B)
    scale, shift = _gnorm_coef(m_ref, gam_ref, bet_ref, cin, bidx)
    a = jnp.maximum(z_ref[...] * scale + shift, 0.0)
    z = jnp.dot(a, w_ref[...], preferred_element_type=jnp.float32) + b_ref[...]
    mo_ref[0, 0:1, :] = jnp.sum(z, axis=0, keepdims=True)
    mo_ref[0, 1:2, :] = jnp.sum(z * z, axis=0, keepdims=True)


def _p4_body(z_ref, m2_ref, g1_ref, be1_ref, w_ref, b_ref, m3_ref, g2_ref,
             be2_ref, out_ref, *, cin, cout):
    bidx = pl.program_id(0) // (NCHUNK // B)
    scale, shift = _gnorm_coef(m2_ref, g1_ref, be1_ref, cin, bidx)
    a = jnp.maximum(z_ref[...] * scale + shift, 0.0)
    z = jnp.dot(a, w_ref[...], preferred_element_type=jnp.float32) + b_ref[...]
    scale2, shift2 = _gnorm_coef(m3_ref, g2_ref, be2_ref, cout, bidx)
    a = jnp.maximum(z * scale2 + shift2, 0.0)
    out_ref[0] = jnp.max(a.reshape(CHUNK_C, NSAMPLE, cout), axis=1)


def _mlp(gath, nxp, W0p, b0r, g0r, be0r, W1t, b1r, g1r, be1r,
         W2t, b2r, g2r, be2r):
    f32 = jnp.float32
    zspec = lambda c: pl.BlockSpec((RCHUNK, c), lambda i: (i, 0))
    mspec_o = lambda c: pl.BlockSpec((1, 2, c), lambda i: (i, 0, 0))
    mspec_i = lambda c: pl.BlockSpec((NCHUNK, 2, c), lambda i: (0, 0, 0))
    wspec = lambda r, c: pl.BlockSpec((r, c), lambda i: (0, 0))

    z1, m1 = pl.pallas_call(
        _p1_body,
        grid=(NCHUNK,),
        out_shape=(jax.ShapeDtypeStruct((ROWS, 64), f32),
                   jax.ShapeDtypeStruct((NCHUNK, 2, 64), f32)),
        in_specs=[zspec(CIN),
                  pl.BlockSpec((CHUNK_C, CIN), lambda i: (i, 0)),
                  wspec(CIN, 64), wspec(1, 64)],
        out_specs=(zspec(64), mspec_o(64)),
    )(gath, nxp, W0p, b0r)

    z2, m2 = pl.pallas_call(
        functools.partial(_p23_body, cin=64, cout=64),
        grid=(NCHUNK,),
        out_shape=(jax.ShapeDtypeStruct((ROWS, 64), f32),
                   jax.ShapeDtypeStruct((NCHUNK, 2, 64), f32)),
        in_specs=[zspec(64), mspec_i(64), wspec(1, 64), wspec(1, 64),
                  wspec(64, 64), wspec(1, 64)],
        out_specs=(zspec(64), mspec_o(64)),
    )(z1, m1, g0r, be0r, W1t, b1r)

    m3 = pl.pallas_call(
        functools.partial(_p3_body, cin=64),
        grid=(NCHUNK,),
        out_shape=jax.ShapeDtypeStruct((NCHUNK, 2, 128), f32),
        in_specs=[zspec(64), mspec_i(64), wspec(1, 64), wspec(1, 64),
                  wspec(64, 128), wspec(1, 128)],
        out_specs=mspec_o(128),
    )(z2, m2, g1r, be1r, W2t, b2r)

    out = pl.pallas_call(
        functools.partial(_p4_body, cin=64, cout=128),
        grid=(NCHUNK,),
        out_shape=jax.ShapeDtypeStruct((B, NPOINT, 128), f32),
        in_specs=[zspec(64), mspec_i(64), wspec(1, 64), wspec(1, 64),
                  wspec(64, 128), wspec(1, 128), mspec_i(128),
                  wspec(1, 128), wspec(1, 128)],
        out_specs=pl.BlockSpec((1, CHUNK_C, 128),
                               lambda i: (i // (NCHUNK // B), i % (NCHUNK // B), 0)),
    )(z2, m2, g1r, be1r, W2t, b2r, m3, g2r, be2r)
    return out


def kernel(xyz, points, W0, b0, g0, be0, W1, b1, g1, be1, W2, b2, g2, be2):
    f32 = jnp.float32
    xyzt = xyz.transpose(0, 2, 1)
    f0 = jax.random.randint(jax.random.key(42), (B,), 0, N)
    f0 = jnp.broadcast_to(f0.astype(jnp.int32)[:, None], (B, 128))

    new_xyz = _fps(xyzt, f0)
    idx = _knn(new_xyz, xyzt)

    table = jnp.concatenate([xyz, points], axis=-1).reshape(B * N, 3 + D)
    table = jnp.pad(table, ((0, 0), (0, CIN - (3 + D))))
    gath = _gather_rows(table, idx.reshape(-1))

    nxp = jnp.pad(new_xyz.reshape(B * NPOINT, 3), ((0, 0), (0, CIN - 3)))
    W0p = jnp.pad(W0.T, ((0, CIN - (3 + D)), (0, 0)))
    out = _mlp(gath, nxp, W0p, b0[None, :], g0[None, :], be0[None, :],
               W1.T, b1[None, :], g1[None, :], be1[None, :],
               W2.T, b2[None, :], g2[None, :], be2[None, :])
    return (new_xyz, out)


# speedup vs baseline: 11.4898x; 1.0008x over previous
import functools

import jax
import jax.numpy as jnp
from jax import lax
from jax.experimental import pallas as pl
from jax.experimental.pallas import tpu as pltpu
from jax.experimental.pallas import tpu_sc as plsc

B, N, D = 4, 4096, 16
NPOINT, NSAMPLE = 1024, 32
GROUPS = 8
EPS = 1e-5
CIN = 128
CHUNK_C = 128
KNN_C = 512
ROWS = B * NPOINT * NSAMPLE
RCHUNK = NPOINT * NSAMPLE // 8
NCHUNK = ROWS // RCHUNK


def _fps_body(xyzt_ref, f0_ref, nxyz_ref):
    XYZ = xyzt_ref[...].reshape(B, 3, 32, 128)
    X = XYZ[:, 0]
    Y = XYZ[:, 1]
    Z = XYZ[:, 2]
    s32 = lax.broadcasted_iota(jnp.int32, (B, 32, 128), 1)
    lane = lax.broadcasted_iota(jnp.int32, (B, 32, 128), 2)
    flat = s32 * 128 + lane

    def step(i, carry):
        dist, fi = carry
        eq = flat == fi
        c3 = jnp.sum(jnp.where(eq[:, None], XYZ, 0.0), axis=(2, 3))
        nxyz_ref[:, pl.ds(i, 1), :] = c3[:, None, :]
        dx = X - c3[:, 0:1, None]
        dy = Y - c3[:, 1:2, None]
        dz = Z - c3[:, 2:3, None]
        d = dx * dx + dy * dy + dz * dz
        dist = jnp.minimum(dist, d)
        m = jnp.max(dist, axis=(1, 2), keepdims=True)
        k = jnp.min(jnp.where(dist == m, flat, jnp.int32(2 ** 30)),
                    axis=(1, 2), keepdims=True)
        return dist, k

    dist0 = jnp.full((B, 32, 128), 1e10, jnp.float32)
    fi0 = f0_ref[:, 0:1].reshape(B, 1, 1)
    lax.fori_loop(0, NPOINT, step, (dist0, fi0))


def _fps(xyzt, f0):
    return pl.pallas_call(
        _fps_body,
        out_shape=jax.ShapeDtypeStruct((B, NPOINT, 3), jnp.float32),
        in_specs=[
            pl.BlockSpec((B, 3, N), lambda: (0, 0, 0)),
            pl.BlockSpec((B, 128), lambda: (0, 0)),
        ],
        out_specs=pl.BlockSpec((B, NPOINT, 3), lambda: (0, 0, 0)),
    )(xyzt, f0)


def _knn_body(nxyz_ref, xyzt_ref, idx_ref):
    b = pl.program_id(0)
    nx = nxyz_ref[0]
    xt = xyzt_ref[0]
    s = jnp.dot(nx.astype(jnp.bfloat16), xt.astype(jnp.bfloat16),
                preferred_element_type=jnp.float32)
    a2 = (nx[:, 0:1] * nx[:, 0:1] + nx[:, 1:2] * nx[:, 1:2]) + nx[:, 2:3] * nx[:, 2:3]
    b2 = (xt[0:1] * xt[0:1] + xt[1:2] * xt[1:2]) + xt[2:3] * xt[2:3]
    d = (-2.0 * s + a2) + b2
    col = lax.broadcasted_iota(jnp.int32, (KNN_C, N), 1)
    base = b * N
    inf = jnp.float32(jnp.inf)
    for s in range(NSAMPLE):
        m = jnp.min(d, axis=1, keepdims=True)
        ci = jnp.min(jnp.where(d == m, col, jnp.int32(2 ** 30)),
                     axis=1, keepdims=True)
        idx_ref[0, :, pl.ds(s, 1)] = ci + base
        d = jnp.where(col == ci, inf, d)


def _knn(nxyz, xyzt):
    return pl.pallas_call(
        _knn_body,
        grid=(B, NPOINT // KNN_C),
        out_shape=jax.ShapeDtypeStruct((B, NPOINT, NSAMPLE), jnp.int32),
        in_specs=[
            pl.BlockSpec((1, KNN_C, 3), lambda b, c: (b, c, 0)),
            pl.BlockSpec((1, 3, N), lambda b, c: (b, 0, 0)),
        ],
        out_specs=pl.BlockSpec((1, KNN_C, NSAMPLE), lambda b, c: (b, c, 0)),
    )(nxyz, xyzt)


@functools.lru_cache(maxsize=1)
def _make_gather():
    info = plsc.get_sparse_core_info()
    nc, ns = info.num_cores, info.num_subcores
    nw = nc * ns
    rpw = ROWS // nw
    gchunk = 128
    niter = rpw // gchunk
    mesh = plsc.VectorSubcoreMesh(core_axis_name="c", subcore_axis_name="s")

    @functools.partial(
        pl.kernel, mesh=mesh,
        out_type=jax.ShapeDtypeStruct((ROWS, CIN), jnp.float32),
        scratch_types=[
            pltpu.VMEM((gchunk,), jnp.int32),
            pltpu.VMEM((gchunk, CIN), jnp.float32),
            pltpu.SemaphoreType.DMA,
        ],
    )
    def gk(table_hbm, idx_hbm, out_hbm, idx_v, rows_v, sem):
        wid = lax.axis_index("s") * nc + lax.axis_index("c")
        base = wid * rpw

        def body(c, _):
            off = base + c * gchunk
            pltpu.sync_copy(idx_hbm.at[pl.ds(off, gchunk)], idx_v)
            pltpu.async_copy(table_hbm.at[idx_v], rows_v, sem).wait()
            pltpu.sync_copy(rows_v, out_hbm.at[pl.ds(off, gchunk)])
            return 0

        lax.fori_loop(0, niter, body, 0)

    return gk


def _gather_rows(table, idx_flat):
    return _make_gather()(table, idx_flat)


def _gnorm_coef(m_ref, gam_ref, bet_ref, cout, bidx):
    per_g = cout // GROUPS
    cnt = jnp.float32(per_g * NPOINT * NSAMPLE)
    sums = jnp.sum(m_ref[pl.ds(bidx * (NCHUNK // B), NCHUNK // B)], axis=0)
    s = sums[0:1, :]
    q = sums[1:2, :]
    ci = lax.broadcasted_iota(jnp.int32, (cout, cout), 0) // per_g
    cj = lax.broadcasted_iota(jnp.int32, (cout, cout), 1) // per_g
    A = (ci == cj).astype(jnp.float32)
    gs = jnp.dot(s, A, preferred_element_type=jnp.float32)
    gq = jnp.dot(q, A, preferred_element_type=jnp.float32)
    mean = gs / cnt
    var = gq / cnt - mean * mean
    inv = lax.rsqrt(var + EPS)
    scale = inv * gam_ref[...]
    shift = bet_ref[...] - mean * scale
    return scale, shift


def _p1_body(g_ref, nx_ref, w_ref, b_ref, z_ref, m_ref):
    g = g_ref[...].reshape(CHUNK_C, NSAMPLE, CIN)
    gn = (g - nx_ref[...][:, None, :]).reshape(RCHUNK, CIN)
    z = jnp.dot(gn, w_ref[...], preferred_element_type=jnp.float32) + b_ref[...]
    z_ref[...] = z
    m_ref[0, 0:1, :] = jnp.sum(z, axis=0, keepdims=True)
    m_ref[0, 1:2, :] = jnp.sum(z * z, axis=0, keepdims=True)


def _p23_body(z_ref, m_ref, gam_ref, bet_ref, w_ref, b_ref, zo_ref, mo_ref,
              *, cin, cout):
    bidx = pl.program_id(0) // (NCHUNK // B)
    scale, shift = _gnorm_coef(m_ref, gam_ref, bet_ref, cin, bidx)
    a = jnp.maximum(z_ref[...] * scale + shift, 0.0)
    z = jnp.dot(a, w_ref[...], preferred_element_type=jnp.float32) + b_ref[...]
    zo_ref[...] = z
    mo_ref[0, 0:1, :] = jnp.sum(z, axis=0, keepdims=True)
    mo_ref[0, 1:2, :] = jnp.sum(z * z, axis=0, keepdims=True)


def _p3_body(z_ref, m_ref, gam_ref, bet_ref, w_ref, b_ref, mo_ref, *, cin):
    bidx = pl.program_id(0) // (NCHUNK // B)
    scale, shift = _gnorm_coef(m_ref, gam_ref, bet_ref, cin, bidx)
    a = jnp.maximum(z_ref[...] * scale + shift, 0.0)
    z = jnp.dot(a, w_ref[...], preferred_element_type=jnp.float32) + b_ref[...]
    mo_ref[0, 0:1, :] = jnp.sum(z, axis=0, keepdims=True)
    mo_ref[0, 1:2, :] = jnp.sum(z * z, axis=0, keepdims=True)


def _p4_body(z_ref, m2_ref, g1_ref, be1_ref, w_ref, b_ref, m3_ref, g2_ref,
             be2_ref, out_ref, *, cin, cout):
    bidx = pl.program_id(0) // (NCHUNK // B)
    scale, shift = _gnorm_coef(m2_ref, g1_ref, be1_ref, cin, bidx)
    a = jnp.maximum(z_ref[...] * scale + shift, 0.0)
    z = jnp.dot(a, w_ref[...], preferred_element_type=jnp.float32) + b_ref[...]
    scale2, shift2 = _gnorm_coef(m3_ref, g2_ref, be2_ref, cout, bidx)
    a = jnp.maximum(z * scale2 + shift2, 0.0)
    out_ref[0] = jnp.max(a.reshape(CHUNK_C, NSAMPLE, cout), axis=1)


def _mlp(gath, nxp, W0p, b0r, g0r, be0r, W1t, b1r, g1r, be1r,
         W2t, b2r, g2r, be2r):
    f32 = jnp.float32
    zspec = lambda c: pl.BlockSpec((RCHUNK, c), lambda i: (i, 0))
    mspec_o = lambda c: pl.BlockSpec((1, 2, c), lambda i: (i, 0, 0))
    mspec_i = lambda c: pl.BlockSpec((NCHUNK, 2, c), lambda i: (0, 0, 0))
    wspec = lambda r, c: pl.BlockSpec((r, c), lambda i: (0, 0))

    z1, m1 = pl.pallas_call(
        _p1_body,
        grid=(NCHUNK,),
        out_shape=(jax.ShapeDtypeStruct((ROWS, 64), f32),
                   jax.ShapeDtypeStruct((NCHUNK, 2, 64), f32)),
        in_specs=[zspec(CIN),
                  pl.BlockSpec((CHUNK_C, CIN), lambda i: (i, 0)),
                  wspec(CIN, 64), wspec(1, 64)],
        out_specs=(zspec(64), mspec_o(64)),
    )(gath, nxp, W0p, b0r)

    z2, m2 = pl.pallas_call(
        functools.partial(_p23_body, cin=64, cout=64),
        grid=(NCHUNK,),
        out_shape=(jax.ShapeDtypeStruct((ROWS, 64), f32),
                   jax.ShapeDtypeStruct((NCHUNK, 2, 64), f32)),
        in_specs=[zspec(64), mspec_i(64), wspec(1, 64), wspec(1, 64),
                  wspec(64, 64), wspec(1, 64)],
        out_specs=(zspec(64), mspec_o(64)),
    )(z1, m1, g0r, be0r, W1t, b1r)

    m3 = pl.pallas_call(
        functools.partial(_p3_body, cin=64),
        grid=(NCHUNK,),
        out_shape=jax.ShapeDtypeStruct((NCHUNK, 2, 128), f32),
        in_specs=[zspec(64), mspec_i(64), wspec(1, 64), wspec(1, 64),
                  wspec(64, 128), wspec(1, 128)],
        out_specs=mspec_o(128),
    )(z2, m2, g1r, be1r, W2t, b2r)

    out = pl.pallas_call(
        functools.partial(_p4_body, cin=64, cout=128),
        grid=(NCHUNK,),
        out_shape=jax.ShapeDtypeStruct((B, NPOINT, 128), f32),
        in_specs=[zspec(64), mspec_i(64), wspec(1, 64), wspec(1, 64),
                  wspec(64, 128), wspec(1, 128), mspec_i(128),
                  wspec(1, 128), wspec(1, 128)],
        out_specs=pl.BlockSpec((1, CHUNK_C, 128),
                               lambda i: (i // (NCHUNK // B), i % (NCHUNK // B), 0)),
    )(z2, m2, g1r, be1r, W2t, b2r, m3, g2r, be2r)
    return out


def kernel(xyz, points, W0, b0, g0, be0, W1, b1, g1, be1, W2, b2, g2, be2):
    f32 = jnp.float32
    xyzt = xyz.transpose(0, 2, 1)
    f0 = jax.random.randint(jax.random.key(42), (B,), 0, N)
    f0 = jnp.broadcast_to(f0.astype(jnp.int32)[:, None], (B, 128))

    new_xyz = _fps(xyzt, f0)
    idx = _knn(new_xyz, xyzt)

    table = jnp.concatenate([xyz, points], axis=-1).reshape(B * N, 3 + D)
    table = jnp.pad(table, ((0, 0), (0, CIN - (3 + D))))
    gath = _gather_rows(table, idx.reshape(-1))

    nxp = jnp.pad(new_xyz.reshape(B * NPOINT, 3), ((0, 0), (0, CIN - 3)))
    W0p = jnp.pad(W0.T, ((0, CIN - (3 + D)), (0, 0)))
    out = _mlp(gath, nxp, W0p, b0[None, :], g0[None, :], be0[None, :],
               W1.T, b1[None, :], g1[None, :], be1[None, :],
               W2.T, b2[None, :], g2[None, :], be2[None, :])
    return (new_xyz, out)
